# baseline (speedup 1.0000x reference)
import math

import jax
import jax.numpy as jnp
from jax.experimental import pallas as pl
from jax.experimental.pallas import tpu as pltpu

_BS = 2048


def _fused_kernel(h_ref, w_ref, depth_ref, mask_ref, loss_ref,
                  keys_ref, sums_ref, *, k):
    lg = jax.lax.dot_general(
        w_ref[...], h_ref[0],
        dimension_numbers=(((1,), (1,)), ((), ())),
        preferred_element_type=jnp.float32)

    i, j = pl.program_id(0), pl.program_id(1)
    nb, bs = keys_ref.shape
    keys_ref[pl.ds(j, 1), :] = jax.lax.bitcast_convert_type(
        jax.nn.sigmoid(lg[0]), jnp.int32)[None, :]

    part = jax.nn.sigmoid(jax.nn.sigmoid(lg))
    step = i * pl.num_programs(1) + j

    @pl.when(step == 0)
    def _init():
        sums_ref[...] = part

    @pl.when(step != 0)
    def _acc():
        sums_ref[...] += part

    @pl.when(j == nb - 1)
    def _select():
        keys = keys_ref[...]
        t = jnp.zeros((1, 1), jnp.int32)
        for q in range(29, -1, -1):
            cand = t | (1 << q)
            cnt = jnp.sum((keys >= cand[0, 0]).astype(jnp.int32))
            t = jnp.where(cnt >= k, cand, t)
        tv = t[0, 0]
        gt = keys > tv
        eq = keys == tv
        need = k - jnp.sum(gt.astype(jnp.int32))
        idx = (jax.lax.broadcasted_iota(jnp.int32, (nb, bs), 0) * bs
               + jax.lax.broadcasted_iota(jnp.int32, (nb, bs), 1))
        m = jnp.zeros((1, 1), jnp.int32)
        s = nb * bs
        for q in range(13, -1, -1):
            cand = m + (1 << q)
            cnt = jnp.sum((eq & (idx < cand[0, 0])).astype(jnp.int32))
            m = jnp.where((cand <= s) & (cnt <= need), cand, m)
        sel = gt | (eq & (idx < m[0, 0]))
        mask_ref[0] = sel
        depth_ref[0] = jnp.where(sel, 3, 1).astype(jnp.int32)

    @pl.when(step == pl.num_programs(0) * pl.num_programs(1) - 1)
    def _loss():
        n = pl.num_programs(0) * pl.num_programs(1) * part.shape[1]
        inv = 1.0 / n
        one = jnp.ones((1, 1), jnp.float32)
        log_t = math.log(1.0 / 3.0)
        acc = one * (3.0 * log_t)
        for r in range(3):
            pr = jnp.sum(sums_ref[r, :]) * inv
            acc = acc - jnp.log(one * pr)
        loss_ref[...] = acc * (1.0 / 9.0)


def kernel(hidden_states, w0, w1, w2):
    b, s, h = hidden_states.shape
    k = max(1, int(s * (1.0 / 3.0)))
    w3 = jnp.stack([w0, w1, w2], axis=0)
    nb = s // _BS

    import functools
    depth, mask, loss = pl.pallas_call(
        functools.partial(_fused_kernel, k=k),
        grid=(b, nb),
        in_specs=[
            pl.BlockSpec((1, _BS, h), lambda i, j: (i, j, 0)),
            pl.BlockSpec((3, h), lambda i, j: (0, 0)),
        ],
        out_specs=[
            pl.BlockSpec((1, nb, _BS), lambda i, j: (i, 0, 0)),
            pl.BlockSpec((1, nb, _BS), lambda i, j: (i, 0, 0)),
            pl.BlockSpec((1, 1), lambda i, j: (0, 0)),
        ],
        out_shape=[
            jax.ShapeDtypeStruct((b, nb, _BS), jnp.int32),
            jax.ShapeDtypeStruct((b, nb, _BS), jnp.bool_),
            jax.ShapeDtypeStruct((1, 1), jnp.float32),
        ],
        scratch_shapes=[
            pltpu.VMEM((nb, _BS), jnp.int32),
            pltpu.VMEM((3, _BS), jnp.float32),
        ],
    )(hidden_states, w3)

    ones = jnp.ones((b, s), dtype=jnp.bool_)
    return (depth.reshape(b, s), loss[0, 0], ones,
            mask.reshape(b, s), mask.reshape(b, s))
